# baseline (speedup 1.0000x reference)
import functools

import jax
import jax.numpy as jnp
from jax import lax
from jax.experimental import pallas as pl
from jax.experimental.pallas import tpu as pltpu
from jax.experimental.pallas import tpu_sc as plsc

ACT = 64
B = 16384

_NC, _NS = 2, 16
NW = _NC * _NS
BPW = B // NW

@functools.cache
def _build_sc_gather():
    mesh = plsc.VectorSubcoreMesh(core_axis_name="c", subcore_axis_name="s")

    @functools.partial(
        pl.kernel,
        mesh=mesh,
        out_type=[
            jax.ShapeDtypeStruct((B, ACT), jnp.float32),
            jax.ShapeDtypeStruct((B,), jnp.float32),
        ],
        scratch_types=[
            pltpu.VMEM((BPW,), jnp.int32),
            pltpu.VMEM((BPW, ACT), jnp.float32),
            pltpu.VMEM((BPW,), jnp.float32),
            pltpu.SemaphoreType.DMA,
            pltpu.SemaphoreType.DMA,
        ],
        compiler_params=pltpu.CompilerParams(use_tc_tiling_on_sc=False),
    )
    def _sc_gather(obs_hbm, pi_hbm, v_hbm, raw_out, v_out,
                   idx_v, rows_v, vvals_v, sem_r, sem_v):
        wid = lax.axis_index("s") * _NC + lax.axis_index("c")
        base = wid * BPW
        pltpu.sync_copy(obs_hbm.at[pl.ds(base, BPW)], idx_v)
        cp_r = pltpu.async_copy(pi_hbm.at[idx_v], rows_v, sem_r)
        cp_v = pltpu.async_copy(v_hbm.at[idx_v], vvals_v, sem_v)
        cp_r.wait()
        cp_v.wait()
        pltpu.sync_copy(rows_v, raw_out.at[pl.ds(base, BPW)])
        pltpu.sync_copy(vvals_v, v_out.at[pl.ds(base, BPW)])

    return _sc_gather


_TC_BLK = 2048


def _tc_body(raw_ref, act_ref, logits_ref, logp_ref):
    raw = raw_ref[...]
    m = jnp.max(raw, axis=-1, keepdims=True)
    e = jnp.exp(raw - m)
    s = jnp.sum(e, axis=-1, keepdims=True)
    lse = m + jnp.log(s)
    logits = raw - lse
    logits_ref[...] = logits
    a = act_ref[...]
    onehot = lax.broadcasted_iota(jnp.int32, (_TC_BLK, ACT), 1) == a
    logp_ref[...] = jnp.sum(jnp.where(onehot, logits, 0.0), axis=-1,
                            keepdims=True)


def _tc_normalize(raw, act2d):
    return pl.pallas_call(
        _tc_body,
        grid=(B // _TC_BLK,),
        in_specs=[
            pl.BlockSpec((_TC_BLK, ACT), lambda i: (i, 0)),
            pl.BlockSpec((_TC_BLK, 1), lambda i: (i, 0)),
        ],
        out_specs=[
            pl.BlockSpec((_TC_BLK, ACT), lambda i: (i, 0)),
            pl.BlockSpec((_TC_BLK, 1), lambda i: (i, 0)),
        ],
        out_shape=[
            jax.ShapeDtypeStruct((B, ACT), jnp.float32),
            jax.ShapeDtypeStruct((B, 1), jnp.float32),
        ],
    )(raw, act2d)


def kernel(obs, act, v_matrix, pi_logit_matrix):
    obs = obs.astype(jnp.int32)
    act2d = act.astype(jnp.int32).reshape(B, 1)
    raw, v = _build_sc_gather()(obs, pi_logit_matrix, v_matrix)
    logits, logp = _tc_normalize(raw, act2d)
    return logits, v, logp.reshape(B)


# speedup vs baseline: 1.6073x; 1.6073x over previous
import functools

import jax
import jax.numpy as jnp
from jax import lax
from jax.experimental import pallas as pl
from jax.experimental.pallas import tpu as pltpu
from jax.experimental.pallas import tpu_sc as plsc

ACT = 64
B = 16384

_NC, _NS = 2, 16
NW = _NC * _NS
BPW = B // NW

_CHUNK = 16


@functools.cache
def _build_sc_gather():
    mesh = plsc.VectorSubcoreMesh(core_axis_name="c", subcore_axis_name="s")

    @functools.partial(
        pl.kernel,
        mesh=mesh,
        out_type=[
            jax.ShapeDtypeStruct((B, ACT), jnp.float32),
            jax.ShapeDtypeStruct((B,), jnp.float32),
        ],
        scratch_types=[
            pltpu.VMEM((BPW,), jnp.int32),
            pltpu.VMEM((BPW, ACT), jnp.float32),
            pltpu.VMEM((BPW,), jnp.float32),
            pltpu.SemaphoreType.DMA,
            pltpu.SemaphoreType.DMA,
        ],
    )
    def _sc_gather(obs_hbm, pi_hbm, v_hbm, raw_out, v_out,
                   idx_v, rows_v, vvals_v, sem_r, sem_v):
        wid = lax.axis_index("s") * _NC + lax.axis_index("c")
        base = wid * BPW
        pltpu.sync_copy(obs_hbm.at[pl.ds(base, BPW)], idx_v)
        cp_v = pltpu.async_copy(v_hbm.at[idx_v], vvals_v, sem_v)

        def chunk(g):
            vec = idx_v[pl.ds(g * _CHUNK, _CHUNK)]
            cps = []
            for j in range(_CHUNK):
                i = g * _CHUNK + j
                cps.append(pltpu.async_copy(
                    pi_hbm.at[vec[j]], rows_v.at[i], sem_r))
            for cp in cps:
                cp.wait()

        pl.loop(0, BPW // _CHUNK)(chunk)

        cp_v.wait()
        pltpu.sync_copy(rows_v, raw_out.at[pl.ds(base, BPW)])
        pltpu.sync_copy(vvals_v, v_out.at[pl.ds(base, BPW)])

    return _sc_gather


_TC_BLK = 2048


def _tc_body(raw_ref, act_ref, logits_ref, logp_ref):
    raw = raw_ref[...]
    m = jnp.max(raw, axis=-1, keepdims=True)
    e = jnp.exp(raw - m)
    s = jnp.sum(e, axis=-1, keepdims=True)
    lse = m + jnp.log(s)
    logits = raw - lse
    logits_ref[...] = logits
    a = act_ref[...]
    onehot = lax.broadcasted_iota(jnp.int32, (_TC_BLK, ACT), 1) == a
    logp_ref[...] = jnp.sum(jnp.where(onehot, logits, 0.0), axis=-1,
                            keepdims=True)


def _tc_normalize(raw, act2d):
    return pl.pallas_call(
        _tc_body,
        grid=(B // _TC_BLK,),
        in_specs=[
            pl.BlockSpec((_TC_BLK, ACT), lambda i: (i, 0)),
            pl.BlockSpec((_TC_BLK, 1), lambda i: (i, 0)),
        ],
        out_specs=[
            pl.BlockSpec((_TC_BLK, ACT), lambda i: (i, 0)),
            pl.BlockSpec((_TC_BLK, 1), lambda i: (i, 0)),
        ],
        out_shape=[
            jax.ShapeDtypeStruct((B, ACT), jnp.float32),
            jax.ShapeDtypeStruct((B, 1), jnp.float32),
        ],
    )(raw, act2d)


def kernel(obs, act, v_matrix, pi_logit_matrix):
    obs = obs.astype(jnp.int32)
    act2d = act.astype(jnp.int32).reshape(B, 1)
    raw, v = _build_sc_gather()(obs, pi_logit_matrix, v_matrix)
    logits, logp = _tc_normalize(raw, act2d)
    return logits, v, logp.reshape(B)
